# baseline (speedup 1.0000x reference)
import functools

import jax
import jax.numpy as jnp
from jax import lax
from jax.experimental import pallas as pl
from jax.experimental.pallas import tpu as pltpu
from jax.experimental.pallas import tpu_sc as plsc

N_NODES = 10000
N_EDGES = 160000
DIM = 256
HALF = 128
N_GRAPHS = 64
BN_EPS = 1e-5

NC = 2
NS = 16
EDGES_PER_TILE = 10000
CHUNK = 125
NCHUNK = EDGES_PER_TILE // CHUNK
ACC_ROWS = 10112
ROWS_PER_TILE = ACC_ROWS // NS

BLK = 10000
GRID = N_NODES // BLK


def _make_sc_agg():
    mesh = plsc.VectorSubcoreMesh(
        core_axis_name="c", subcore_axis_name="s", num_cores=NC, num_subcores=NS
    )

    @functools.partial(
        pl.kernel,
        out_type=[
            jax.ShapeDtypeStruct((ACC_ROWS, HALF), jnp.float32),
            jax.ShapeDtypeStruct((ACC_ROWS, HALF), jnp.float32),
        ],
        mesh=mesh,
        scratch_types=[
            pltpu.VMEM((NCHUNK, CHUNK), jnp.int32),
            pltpu.VMEM((NCHUNK, CHUNK), jnp.int32),
            pltpu.VMEM((CHUNK, HALF), jnp.float32),
            pltpu.VMEM_SHARED((ACC_ROWS, HALF), jnp.float32),
            pltpu.SemaphoreType.DMA,
        ],
    )
    def sc_agg(hlo_hbm, hhi_hbm, src_hbm, dst_hbm, zero_hbm, alo_hbm, ahi_hbm,
               src_v, dst_v, rows_v, acc_sh, sem):
        c = lax.axis_index("c")
        sid = lax.axis_index("s")
        pltpu.sync_copy(zero_hbm,
                        acc_sh.at[pl.ds(sid * ROWS_PER_TILE, ROWS_PER_TILE)])
        pltpu.sync_copy(src_hbm.at[sid], src_v)
        pltpu.sync_copy(dst_hbm.at[sid], dst_v)
        plsc.subcore_barrier()

        def run(h_hbm, out_hbm):
            @pl.loop(0, NCHUNK)
            def _(j):
                pltpu.async_copy(h_hbm.at[src_v.at[j]], rows_v, sem).wait()
                pltpu.sync_copy(rows_v, acc_sh.at[dst_v.at[j]], add=True)

            plsc.subcore_barrier()
            sl = pl.ds(sid * ROWS_PER_TILE, ROWS_PER_TILE)
            pltpu.sync_copy(acc_sh.at[sl], out_hbm.at[sl])

        @pl.when(c == 0)
        def _():
            run(hlo_hbm, alo_hbm)

        @pl.when(c == 1)
        def _():
            run(hhi_hbm, ahi_hbm)

    return sc_agg


_SC_AGG_CACHE = []


def _sc_agg(*args):
    if not _SC_AGG_CACHE:
        _SC_AGG_CACHE.append(_make_sc_agg())
    return _SC_AGG_CACHE[0](*args)


def _mlp_stats_body(hlo, hhi, alo, ahi, w1, b1, w2, b2, m_out, ssum, ssq):
    i = pl.program_id(0)
    h = jnp.concatenate([hlo[...] + alo[...], hhi[...] + ahi[...]], axis=1)
    z = jnp.maximum(
        jnp.dot(h, w1[...], preferred_element_type=jnp.float32) + b1[...], 0.0
    )
    m = jnp.dot(z, w2[...], preferred_element_type=jnp.float32) + b2[...]
    m = jnp.maximum(m, 0.0)
    m_out[...] = m
    cs = jnp.sum(m, axis=0, keepdims=True)
    cq = jnp.sum(m * m, axis=0, keepdims=True)

    @pl.when(i == 0)
    def _():
        ssum[...] = cs
        ssq[...] = cq

    @pl.when(i > 0)
    def _():
        ssum[...] += cs
        ssq[...] += cq


def _mlp_stats(hlo, hhi, alo, ahi, w1, b1, w2, b2):
    half_in = pl.BlockSpec((BLK, HALF), lambda i: (i, 0))
    full_w = pl.BlockSpec((DIM, DIM), lambda i: (0, 0))
    row = pl.BlockSpec((1, DIM), lambda i: (0, 0))
    return pl.pallas_call(
        _mlp_stats_body,
        grid=(GRID,),
        in_specs=[half_in, half_in, half_in, half_in, full_w, row, full_w, row],
        out_specs=[
            pl.BlockSpec((BLK, DIM), lambda i: (i, 0)),
            row,
            row,
        ],
        out_shape=[
            jax.ShapeDtypeStruct((N_NODES, DIM), jnp.float32),
            jax.ShapeDtypeStruct((1, DIM), jnp.float32),
            jax.ShapeDtypeStruct((1, DIM), jnp.float32),
        ],
    )(hlo, hhi, alo, ahi, w1, b1, w2, b2)


def _norm_pool_body(m_ref, ssum, ssq, g_ref, be_ref, batch_ref, *rest):
    hlo_out, hhi_out, pool_out, xn_out = rest[-4:]
    i = pl.program_id(0)
    inv_n = 1.0 / N_NODES
    mean = ssum[...] * inv_n
    var = ssq[...] * inv_n - mean * mean
    scale = g_ref[...] * lax.rsqrt(var + BN_EPS)
    shift = be_ref[...] - mean * scale
    hq = m_ref[...] * scale + shift
    hlo_out[...] = hq[:, :HALF]
    hhi_out[...] = hq[:, HALF:]
    xn_out[...] = hq
    bb = batch_ref[0, 0, :]
    onehot = (bb[None, :] == lax.broadcasted_iota(jnp.int32, (N_GRAPHS, BLK), 0))
    contrib = jnp.dot(onehot.astype(jnp.float32), hq,
                      preferred_element_type=jnp.float32)

    @pl.when(i == 0)
    def _():
        pool_out[...] = contrib

    @pl.when(i > 0)
    def _():
        pool_out[...] += contrib


def _norm_pool(m, ssum, ssq, g, be, batch3d, layer, xn_prev):
    row = pl.BlockSpec((1, DIM), lambda i: (0, 0))
    in_specs = [
        pl.BlockSpec((BLK, DIM), lambda i: (i, 0)),
        row, row, row, row,
        pl.BlockSpec((1, 1, BLK), lambda i: (i, 0, 0)),
    ]
    out_specs = [
        pl.BlockSpec((BLK, HALF), lambda i: (i, 0)),
        pl.BlockSpec((BLK, HALF), lambda i: (i, 0)),
        pl.BlockSpec((N_GRAPHS, DIM), lambda i: (0, 0)),
        pl.BlockSpec((BLK, DIM), lambda i, layer=layer: (i, layer)),
    ]
    out_shape = [
        jax.ShapeDtypeStruct((N_NODES, HALF), jnp.float32),
        jax.ShapeDtypeStruct((N_NODES, HALF), jnp.float32),
        jax.ShapeDtypeStruct((N_GRAPHS, DIM), jnp.float32),
        jax.ShapeDtypeStruct((N_NODES, 3 * DIM), jnp.float32),
    ]
    args = [m, ssum, ssq, g, be, batch3d]
    aliases = {}
    if xn_prev is not None:
        in_specs.append(pl.BlockSpec(memory_space=pl.ANY))
        args.append(xn_prev)
        aliases = {6: 3}
    return pl.pallas_call(
        _norm_pool_body,
        grid=(GRID,),
        in_specs=in_specs,
        out_specs=out_specs,
        out_shape=out_shape,
        input_output_aliases=aliases,
    )(*args)


def kernel(x, edge_index, batch,
           W1_0, b1_0, W2_0, b2_0, g_0, be_0,
           W1_1, b1_1, W2_1, b2_1, g_1, be_1,
           W1_2, b1_2, W2_2, b2_2, g_2, be_2):
    params = [(W1_0, b1_0, W2_0, b2_0, g_0, be_0),
              (W1_1, b1_1, W2_1, b2_1, g_1, be_1),
              (W1_2, b1_2, W2_2, b2_2, g_2, be_2)]
    src_r = edge_index[0].reshape(NS, NCHUNK, CHUNK)
    dst_r = edge_index[1].reshape(NS, NCHUNK, CHUNK)
    zeros = jnp.zeros((ROWS_PER_TILE, HALF), jnp.float32)
    batch3d = batch.reshape(GRID, 1, BLK)

    h_lo = x[:, :HALF]
    h_hi = x[:, HALF:]
    pools = []
    xn = None
    for layer, (w1, b1, w2, b2, g, be) in enumerate(params):
        agg_lo, agg_hi = _sc_agg(h_lo, h_hi, src_r, dst_r, zeros)
        m, ssum, ssq = _mlp_stats(h_lo, h_hi, agg_lo, agg_hi,
                                  w1, b1.reshape(1, DIM), w2, b2.reshape(1, DIM))
        h_lo, h_hi, pool, xn = _norm_pool(m, ssum, ssq, g.reshape(1, DIM),
                                          be.reshape(1, DIM), batch3d, layer, xn)
        pools.append(pool)

    x_g = jnp.concatenate(pools, axis=1)
    return (x_g, xn)


# speedup vs baseline: 1.0204x; 1.0204x over previous
import functools

import jax
import jax.numpy as jnp
from jax import lax
from jax.experimental import pallas as pl
from jax.experimental.pallas import tpu as pltpu
from jax.experimental.pallas import tpu_sc as plsc

N_NODES = 10000
N_EDGES = 160000
DIM = 256
HALF = 128
N_GRAPHS = 64
BN_EPS = 1e-5

NC = 2
NS = 16
EDGES_PER_TILE = 10000
CHUNK = 125
NCHUNK = EDGES_PER_TILE // CHUNK
ACC_ROWS = 10112
ROWS_PER_TILE = ACC_ROWS // NS

BLK = 5000
GRID = N_NODES // BLK


def _make_sc_agg():
    mesh = plsc.VectorSubcoreMesh(
        core_axis_name="c", subcore_axis_name="s", num_cores=NC, num_subcores=NS
    )

    @functools.partial(
        pl.kernel,
        out_type=[
            jax.ShapeDtypeStruct((ACC_ROWS, HALF), jnp.float32),
            jax.ShapeDtypeStruct((ACC_ROWS, HALF), jnp.float32),
        ],
        mesh=mesh,
        scratch_types=[
            pltpu.VMEM((NCHUNK, CHUNK), jnp.int32),
            pltpu.VMEM((NCHUNK, CHUNK), jnp.int32),
            pltpu.VMEM((CHUNK, HALF), jnp.float32),
            pltpu.VMEM_SHARED((ACC_ROWS, HALF), jnp.float32),
            pltpu.SemaphoreType.DMA,
        ],
    )
    def sc_agg(hlo_hbm, hhi_hbm, src_hbm, dst_hbm, zero_hbm, alo_hbm, ahi_hbm,
               src_v, dst_v, rows_v, acc_sh, sem):
        c = lax.axis_index("c")
        sid = lax.axis_index("s")
        pltpu.sync_copy(zero_hbm,
                        acc_sh.at[pl.ds(sid * ROWS_PER_TILE, ROWS_PER_TILE)])
        pltpu.sync_copy(src_hbm.at[sid], src_v)
        pltpu.sync_copy(dst_hbm.at[sid], dst_v)
        plsc.subcore_barrier()

        def run(h_hbm, out_hbm):
            @pl.loop(0, NCHUNK)
            def _(j):
                pltpu.async_copy(h_hbm.at[src_v.at[j]], rows_v, sem).wait()
                pltpu.sync_copy(rows_v, acc_sh.at[dst_v.at[j]], add=True)

            plsc.subcore_barrier()
            sl = pl.ds(sid * ROWS_PER_TILE, ROWS_PER_TILE)
            pltpu.sync_copy(acc_sh.at[sl], out_hbm.at[sl])

        @pl.when(c == 0)
        def _():
            run(hlo_hbm, alo_hbm)

        @pl.when(c == 1)
        def _():
            run(hhi_hbm, ahi_hbm)

    return sc_agg


_SC_AGG_CACHE = []


def _sc_agg(*args):
    if not _SC_AGG_CACHE:
        _SC_AGG_CACHE.append(_make_sc_agg())
    return _SC_AGG_CACHE[0](*args)


def _mlp_stats_body(hlo, hhi, alo, ahi, w1, b1, w2, b2, m_out, ssum, ssq):
    i = pl.program_id(0)
    h = jnp.concatenate([hlo[...] + alo[...], hhi[...] + ahi[...]], axis=1)
    z = jnp.maximum(
        jnp.dot(h, w1[...], preferred_element_type=jnp.float32) + b1[...], 0.0
    )
    m = jnp.dot(z, w2[...], preferred_element_type=jnp.float32) + b2[...]
    m = jnp.maximum(m, 0.0)
    m_out[...] = m
    cs = jnp.sum(m, axis=0, keepdims=True)
    cq = jnp.sum(m * m, axis=0, keepdims=True)

    @pl.when(i == 0)
    def _():
        ssum[...] = cs
        ssq[...] = cq

    @pl.when(i > 0)
    def _():
        ssum[...] += cs
        ssq[...] += cq


def _mlp_stats(hlo, hhi, alo, ahi, w1, b1, w2, b2):
    half_in = pl.BlockSpec((BLK, HALF), lambda i: (i, 0))
    full_w = pl.BlockSpec((DIM, DIM), lambda i: (0, 0))
    row = pl.BlockSpec((1, DIM), lambda i: (0, 0))
    return pl.pallas_call(
        _mlp_stats_body,
        grid=(GRID,),
        in_specs=[half_in, half_in, half_in, half_in, full_w, row, full_w, row],
        out_specs=[
            pl.BlockSpec((BLK, DIM), lambda i: (i, 0)),
            row,
            row,
        ],
        out_shape=[
            jax.ShapeDtypeStruct((N_NODES, DIM), jnp.float32),
            jax.ShapeDtypeStruct((1, DIM), jnp.float32),
            jax.ShapeDtypeStruct((1, DIM), jnp.float32),
        ],
    )(hlo, hhi, alo, ahi, w1, b1, w2, b2)


def _norm_pool_body(m_ref, ssum, ssq, g_ref, be_ref, batch_ref, *rest):
    hlo_out, hhi_out, pool_out, xn_out = rest[-4:]
    i = pl.program_id(0)
    inv_n = 1.0 / N_NODES
    mean = ssum[...] * inv_n
    var = ssq[...] * inv_n - mean * mean
    scale = g_ref[...] * lax.rsqrt(var + BN_EPS)
    shift = be_ref[...] - mean * scale
    hq = m_ref[...] * scale + shift
    hlo_out[...] = hq[:, :HALF]
    hhi_out[...] = hq[:, HALF:]
    xn_out[...] = hq
    bb = batch_ref[0, 0, :]
    onehot = (bb[None, :] == lax.broadcasted_iota(jnp.int32, (N_GRAPHS, BLK), 0))
    contrib = jnp.dot(onehot.astype(jnp.float32), hq,
                      preferred_element_type=jnp.float32)

    @pl.when(i == 0)
    def _():
        pool_out[...] = contrib

    @pl.when(i > 0)
    def _():
        pool_out[...] += contrib


def _norm_pool(m, ssum, ssq, g, be, batch3d, layer, xn_prev):
    row = pl.BlockSpec((1, DIM), lambda i: (0, 0))
    in_specs = [
        pl.BlockSpec((BLK, DIM), lambda i: (i, 0)),
        row, row, row, row,
        pl.BlockSpec((1, 1, BLK), lambda i: (i, 0, 0)),
    ]
    out_specs = [
        pl.BlockSpec((BLK, HALF), lambda i: (i, 0)),
        pl.BlockSpec((BLK, HALF), lambda i: (i, 0)),
        pl.BlockSpec((N_GRAPHS, DIM), lambda i: (0, 0)),
        pl.BlockSpec((BLK, DIM), lambda i, layer=layer: (i, layer)),
    ]
    out_shape = [
        jax.ShapeDtypeStruct((N_NODES, HALF), jnp.float32),
        jax.ShapeDtypeStruct((N_NODES, HALF), jnp.float32),
        jax.ShapeDtypeStruct((N_GRAPHS, DIM), jnp.float32),
        jax.ShapeDtypeStruct((N_NODES, 3 * DIM), jnp.float32),
    ]
    args = [m, ssum, ssq, g, be, batch3d]
    aliases = {}
    if xn_prev is not None:
        in_specs.append(pl.BlockSpec(memory_space=pl.ANY))
        args.append(xn_prev)
        aliases = {6: 3}
    return pl.pallas_call(
        _norm_pool_body,
        grid=(GRID,),
        in_specs=in_specs,
        out_specs=out_specs,
        out_shape=out_shape,
        input_output_aliases=aliases,
    )(*args)


def kernel(x, edge_index, batch,
           W1_0, b1_0, W2_0, b2_0, g_0, be_0,
           W1_1, b1_1, W2_1, b2_1, g_1, be_1,
           W1_2, b1_2, W2_2, b2_2, g_2, be_2):
    params = [(W1_0, b1_0, W2_0, b2_0, g_0, be_0),
              (W1_1, b1_1, W2_1, b2_1, g_1, be_1),
              (W1_2, b1_2, W2_2, b2_2, g_2, be_2)]
    src_r = edge_index[0].reshape(NS, NCHUNK, CHUNK)
    dst_r = edge_index[1].reshape(NS, NCHUNK, CHUNK)
    zeros = jnp.zeros((ROWS_PER_TILE, HALF), jnp.float32)
    batch3d = batch.reshape(GRID, 1, BLK)

    h_lo = x[:, :HALF]
    h_hi = x[:, HALF:]
    pools = []
    xn = None
    for layer, (w1, b1, w2, b2, g, be) in enumerate(params):
        agg_lo, agg_hi = _sc_agg(h_lo, h_hi, src_r, dst_r, zeros)
        m, ssum, ssq = _mlp_stats(h_lo, h_hi, agg_lo, agg_hi,
                                  w1, b1.reshape(1, DIM), w2, b2.reshape(1, DIM))
        h_lo, h_hi, pool, xn = _norm_pool(m, ssum, ssq, g.reshape(1, DIM),
                                          be.reshape(1, DIM), batch3d, layer, xn)
        pools.append(pool)

    x_g = jnp.concatenate(pools, axis=1)
    return (x_g, xn)
